# baseline (speedup 1.0000x reference)
import functools

import jax
import jax.numpy as jnp
from jax import lax
from jax.experimental import pallas as pl
from jax.experimental.pallas import tpu as pltpu
from jax.experimental.pallas import tpu_sc as plsc

B = 8
N = 100000
M = 128
NC = 2
NS = 16
CHUNKS = 4
C = 25024
NEG = float("-inf")


def _build(b, n, m, c_pad, interpret=False):
    chunks = CHUNKS
    npad = chunks * c_pad
    nslices = c_pad // 16
    mesh = plsc.VectorSubcoreMesh(
        core_axis_name="c", subcore_axis_name="s",
        num_cores=NC, num_subcores=NS)

    @functools.partial(
        pl.kernel,
        out_type=jax.ShapeDtypeStruct((b * m,), jnp.int32),
        mesh=mesh,
        compiler_params=pltpu.CompilerParams(use_tc_tiling_on_sc=False,
                                             needs_layout_passes=False),
        interpret=interpret,
        scratch_types=[
            pltpu.VMEM((3, c_pad), jnp.float32),
            pltpu.VMEM((c_pad,), jnp.float32),
            pltpu.VMEM((m,), jnp.int32),
            pltpu.VMEM((16,), jnp.float32),
            pltpu.VMEM((16,), jnp.int32),
            pltpu.VMEM((chunks, 16), jnp.float32),
            pltpu.VMEM_SHARED((NS, 16), jnp.float32),
        ],
    )
    def fps(pts_hbm, out_hbm, pts_v, md_v, out_v, pubv, bidxbuf,
            redv_l, shv):
        c = lax.axis_index("c")
        s = lax.axis_index("s")
        batch = c * (b // NC) + s // chunks
        bl = s // chunks
        chunk = s % chunks
        base = chunk * c_pad

        lanes = lax.iota(jnp.int32, 16)
        lane0 = lanes == 0
        zero16 = jnp.zeros((16,), jnp.int32)

        for k in range(3):
            flatbase = (k * b + batch) * npad + base
            pltpu.sync_copy(pts_hbm.at[pl.ds(flatbase, c_pad)], pts_v.at[k])

        @plsc.parallel_loop(0, c_pad, 16, unroll=4)
        def _init(off):
            gidx = base + off + lanes
            md_v[pl.ds(off, 16)] = jnp.where(
                gidx < n, jnp.float32(jnp.inf), NEG)

        rowsel = jnp.clip(lanes - 1, 0, 2)

        def publish_combine(mval, coords, gidxv):
            row = jnp.where(lanes == 0, jnp.full((16,), mval, jnp.float32),
                            coords)
            row = jnp.where(lanes == 4, plsc.bitcast(gidxv, jnp.float32),
                            row)
            pubv[...] = row
            pltpu.sync_copy(pubv, shv.at[s])
            plsc.subcore_barrier()
            pltpu.sync_copy(shv.at[pl.ds(bl * chunks, chunks)], redv_l)
            rv0 = redv_l[0]
            ri0 = plsc.bitcast(rv0, jnp.int32)
            vb = rv0[0]
            gb = ri0[4]
            cxb = rv0[1]
            cyb = rv0[2]
            czb = rv0[3]
            for r in range(1, chunks):
                rvr = redv_l[r]
                rir = plsc.bitcast(rvr, jnp.int32)
                vr = rvr[0]
                p = vr > vb
                vb = jnp.where(p, vr, vb)
                gb = jnp.where(p, rir[4], gb)
                cxb = jnp.where(p, rvr[1], cxb)
                cyb = jnp.where(p, rvr[2], cyb)
                czb = jnp.where(p, rvr[3], czb)
            return gb, cxb, cyb, czb

        mval0 = jnp.where(chunk == 0, jnp.float32(1.0), NEG)
        coords0 = plsc.load_gather(pts_v, [rowsel, zero16])
        init_carry = publish_combine(mval0, coords0, zero16 + base)

        def step(t, carry):
            g, cx, cy, cz = carry
            gv = jnp.full((16,), g, jnp.int32)
            plsc.store_scatter(out_v, [jnp.full((16,), t, jnp.int32)], gv,
                               mask=lane0)
            lloc = g - base
            own = (lloc >= 0) & (lloc < c_pad)
            lclamp = jnp.clip(lloc, 0, c_pad - 1)
            plsc.store_scatter(md_v, [jnp.full((16,), lclamp, jnp.int32)],
                               jnp.full((16,), -1.0, jnp.float32),
                               mask=lane0 & jnp.full((16,), own))
            cxv = jnp.full((16,), cx, jnp.float32)
            cyv = jnp.full((16,), cy, jnp.float32)
            czv = jnp.full((16,), cz, jnp.float32)

            @plsc.parallel_loop(
                0, c_pad, 16, unroll=4,
                carry=(jnp.full((16,), NEG, jnp.float32), lanes))
            def scan(off, car):
                best, bidx = car
                x = pts_v[0, pl.ds(off, 16)]
                y = pts_v[1, pl.ds(off, 16)]
                z = pts_v[2, pl.ds(off, 16)]
                dx = x - cxv
                dy = y - cyv
                dz = z - czv
                d = dx * dx + dy * dy + dz * dz
                md = md_v[pl.ds(off, 16)]
                mm = jnp.minimum(md, d)
                md_v[pl.ds(off, 16)] = mm
                upd = mm > best
                best = jnp.where(upd, mm, best)
                bidx = jnp.where(upd, off + lanes, bidx)
                return best, bidx

            best, bidx = scan

            mx = jnp.max(best)
            eq = best == jnp.full((16,), mx, jnp.float32)
            lane = plsc.all_reduce_ffs(eq)
            lanev = (lane if lane.shape == (16,)
                     else jnp.full((16,), lane, jnp.int32))
            bidxbuf[...] = bidx
            lbest = plsc.load_gather(bidxbuf, [lanev])
            coordsb = plsc.load_gather(pts_v, [rowsel, lbest])
            return publish_combine(mx, coordsb, lbest + base)

        lax.fori_loop(0, m, step, init_carry)

        pltpu.sync_copy(out_v, out_hbm.at[pl.ds(batch * m, m)])

    def wrapper(points):
        pts = jnp.transpose(points, (2, 0, 1))
        pts = jnp.pad(pts, ((0, 0), (0, 0), (0, npad - n)))
        return fps(pts.reshape(-1)).reshape(b, m)

    return wrapper


_fps_cache = None


def kernel(points):
    global _fps_cache
    if _fps_cache is None:
        _fps_cache = _build(B, N, M, C)
    return _fps_cache(points)


# speedup vs baseline: 1.0179x; 1.0179x over previous
import functools

import jax
import jax.numpy as jnp
from jax import lax
from jax.experimental import pallas as pl
from jax.experimental.pallas import tpu as pltpu
from jax.experimental.pallas import tpu_sc as plsc

B = 8
N = 100000
M = 128
NC = 2
NS = 16
CHUNKS = 4
C = 25024
NEG = float("-inf")


def _build(b, n, m, c_pad, interpret=False):
    chunks = CHUNKS
    npad = chunks * c_pad
    nslices = c_pad // 16
    mesh = plsc.VectorSubcoreMesh(
        core_axis_name="c", subcore_axis_name="s",
        num_cores=NC, num_subcores=NS)

    @functools.partial(
        pl.kernel,
        out_type=jax.ShapeDtypeStruct((b * m,), jnp.int32),
        mesh=mesh,
        compiler_params=pltpu.CompilerParams(use_tc_tiling_on_sc=False,
                                             needs_layout_passes=False),
        interpret=interpret,
        scratch_types=[
            pltpu.VMEM((3, c_pad), jnp.float32),
            pltpu.VMEM((c_pad,), jnp.float32),
            pltpu.VMEM((m,), jnp.int32),
            pltpu.VMEM((16,), jnp.float32),
            pltpu.VMEM((16,), jnp.int32),
            pltpu.VMEM((chunks, 16), jnp.float32),
            pltpu.VMEM_SHARED((NS, 16), jnp.float32),
        ],
    )
    def fps(pts_hbm, out_hbm, pts_v, md_v, out_v, pubv, bidxbuf,
            redv_l, shv):
        c = lax.axis_index("c")
        s = lax.axis_index("s")
        batch = c * (b // NC) + s // chunks
        bl = s // chunks
        chunk = s % chunks
        base = chunk * c_pad

        lanes = lax.iota(jnp.int32, 16)
        lane0 = lanes == 0
        zero16 = jnp.zeros((16,), jnp.int32)

        for k in range(3):
            flatbase = (k * b + batch) * npad + base
            pltpu.sync_copy(pts_hbm.at[pl.ds(flatbase, c_pad)], pts_v.at[k])

        @plsc.parallel_loop(0, c_pad, 16, unroll=4)
        def _init(off):
            gidx = base + off + lanes
            md_v[pl.ds(off, 16)] = jnp.where(
                gidx < n, jnp.float32(jnp.inf), NEG)

        rowsel = jnp.clip(lanes - 1, 0, 2)

        def publish_combine(mval, coords, gidxv):
            row = jnp.where(lanes == 0, jnp.full((16,), mval, jnp.float32),
                            coords)
            row = jnp.where(lanes == 4, plsc.bitcast(gidxv, jnp.float32),
                            row)
            pubv[...] = row
            pltpu.sync_copy(pubv, shv.at[s])
            plsc.subcore_barrier()
            pltpu.sync_copy(shv.at[pl.ds(bl * chunks, chunks)], redv_l)
            rv0 = redv_l[0]
            ri0 = plsc.bitcast(rv0, jnp.int32)
            vb = rv0[0]
            gb = ri0[4]
            cxb = rv0[1]
            cyb = rv0[2]
            czb = rv0[3]
            for r in range(1, chunks):
                rvr = redv_l[r]
                rir = plsc.bitcast(rvr, jnp.int32)
                vr = rvr[0]
                p = vr > vb
                vb = jnp.where(p, vr, vb)
                gb = jnp.where(p, rir[4], gb)
                cxb = jnp.where(p, rvr[1], cxb)
                cyb = jnp.where(p, rvr[2], cyb)
                czb = jnp.where(p, rvr[3], czb)
            return gb, cxb, cyb, czb

        mval0 = jnp.where(chunk == 0, jnp.float32(1.0), NEG)
        coords0 = plsc.load_gather(pts_v, [rowsel, zero16])
        init_carry = publish_combine(mval0, coords0, zero16 + base)

        def step(t, carry):
            g, cx, cy, cz = carry
            gv = jnp.full((16,), g, jnp.int32)
            plsc.store_scatter(out_v, [jnp.full((16,), t, jnp.int32)], gv,
                               mask=lane0)
            lloc = g - base
            own = (lloc >= 0) & (lloc < c_pad)
            lclamp = jnp.clip(lloc, 0, c_pad - 1)
            plsc.store_scatter(md_v, [jnp.full((16,), lclamp, jnp.int32)],
                               jnp.full((16,), -1.0, jnp.float32),
                               mask=lane0 & jnp.full((16,), own))
            cxv = jnp.full((16,), cx, jnp.float32)
            cyv = jnp.full((16,), cy, jnp.float32)
            czv = jnp.full((16,), cz, jnp.float32)

            @plsc.parallel_loop(
                0, c_pad, 16, unroll=8,
                carry=(jnp.full((16,), NEG, jnp.float32), lanes))
            def scan(off, car):
                best, bidx = car
                x = pts_v[0, pl.ds(off, 16)]
                y = pts_v[1, pl.ds(off, 16)]
                z = pts_v[2, pl.ds(off, 16)]
                dx = x - cxv
                dy = y - cyv
                dz = z - czv
                d = dx * dx + dy * dy + dz * dz
                md = md_v[pl.ds(off, 16)]
                mm = jnp.minimum(md, d)
                md_v[pl.ds(off, 16)] = mm
                upd = mm > best
                best = jnp.where(upd, mm, best)
                bidx = jnp.where(upd, off + lanes, bidx)
                return best, bidx

            best, bidx = scan

            mx = jnp.max(best)
            eq = best == jnp.full((16,), mx, jnp.float32)
            lane = plsc.all_reduce_ffs(eq)
            lanev = (lane if lane.shape == (16,)
                     else jnp.full((16,), lane, jnp.int32))
            bidxbuf[...] = bidx
            lbest = plsc.load_gather(bidxbuf, [lanev])
            coordsb = plsc.load_gather(pts_v, [rowsel, lbest])
            return publish_combine(mx, coordsb, lbest + base)

        lax.fori_loop(0, m, step, init_carry)

        pltpu.sync_copy(out_v, out_hbm.at[pl.ds(batch * m, m)])

    def wrapper(points):
        pts = jnp.transpose(points, (2, 0, 1))
        pts = jnp.pad(pts, ((0, 0), (0, 0), (0, npad - n)))
        return fps(pts.reshape(-1)).reshape(b, m)

    return wrapper


_fps_cache = None


def kernel(points):
    global _fps_cache
    if _fps_cache is None:
        _fps_cache = _build(B, N, M, C)
    return _fps_cache(points)


# speedup vs baseline: 1.0212x; 1.0033x over previous
import functools

import jax
import jax.numpy as jnp
from jax import lax
from jax.experimental import pallas as pl
from jax.experimental.pallas import tpu as pltpu
from jax.experimental.pallas import tpu_sc as plsc

B = 8
N = 100000
M = 128
NC = 2
NS = 16
CHUNKS = 4
C = 25024
NEG = float("-inf")


def _build(b, n, m, c_pad, interpret=False):
    chunks = CHUNKS
    npad = chunks * c_pad
    nslices = c_pad // 16
    mesh = plsc.VectorSubcoreMesh(
        core_axis_name="c", subcore_axis_name="s",
        num_cores=NC, num_subcores=NS)

    @functools.partial(
        pl.kernel,
        out_type=jax.ShapeDtypeStruct((b * m,), jnp.int32),
        mesh=mesh,
        compiler_params=pltpu.CompilerParams(use_tc_tiling_on_sc=False,
                                             needs_layout_passes=False),
        interpret=interpret,
        scratch_types=[
            pltpu.VMEM((3, c_pad), jnp.float32),
            pltpu.VMEM((c_pad,), jnp.float32),
            pltpu.VMEM((m,), jnp.int32),
            pltpu.VMEM((16,), jnp.float32),
            pltpu.VMEM((16,), jnp.int32),
            pltpu.VMEM((chunks, 16), jnp.float32),
            pltpu.VMEM_SHARED((NS, 16), jnp.float32),
        ],
    )
    def fps(pts_hbm, out_hbm, pts_v, md_v, out_v, pubv, bidxbuf,
            redv_l, shv):
        c = lax.axis_index("c")
        s = lax.axis_index("s")
        batch = c * (b // NC) + s // chunks
        bl = s // chunks
        chunk = s % chunks
        base = chunk * c_pad

        lanes = lax.iota(jnp.int32, 16)
        lane0 = lanes == 0
        zero16 = jnp.zeros((16,), jnp.int32)

        for k in range(3):
            flatbase = (k * b + batch) * npad + base
            pltpu.sync_copy(pts_hbm.at[pl.ds(flatbase, c_pad)], pts_v.at[k])

        @plsc.parallel_loop(0, c_pad, 16, unroll=4)
        def _init(off):
            gidx = base + off + lanes
            md_v[pl.ds(off, 16)] = jnp.where(
                gidx < n, jnp.float32(jnp.inf), NEG)

        rowsel = jnp.clip(lanes - 1, 0, 2)

        def publish_combine(mval, coords, gidxv):
            row = jnp.where(lanes == 0, jnp.full((16,), mval, jnp.float32),
                            coords)
            row = jnp.where(lanes == 4, plsc.bitcast(gidxv, jnp.float32),
                            row)
            pubv[...] = row
            pltpu.sync_copy(pubv, shv.at[s])
            plsc.subcore_barrier()
            pltpu.sync_copy(shv.at[pl.ds(bl * chunks, chunks)], redv_l)
            rv0 = redv_l[0]
            ri0 = plsc.bitcast(rv0, jnp.int32)
            vb = rv0[0]
            gb = ri0[4]
            cxb = rv0[1]
            cyb = rv0[2]
            czb = rv0[3]
            for r in range(1, chunks):
                rvr = redv_l[r]
                rir = plsc.bitcast(rvr, jnp.int32)
                vr = rvr[0]
                p = vr > vb
                vb = jnp.where(p, vr, vb)
                gb = jnp.where(p, rir[4], gb)
                cxb = jnp.where(p, rvr[1], cxb)
                cyb = jnp.where(p, rvr[2], cyb)
                czb = jnp.where(p, rvr[3], czb)
            return gb, cxb, cyb, czb

        mval0 = jnp.where(chunk == 0, jnp.float32(1.0), NEG)
        coords0 = plsc.load_gather(pts_v, [rowsel, zero16])
        init_carry = publish_combine(mval0, coords0, zero16 + base)

        def step(t, carry):
            g, cx, cy, cz = carry
            gv = jnp.full((16,), g, jnp.int32)
            plsc.store_scatter(out_v, [jnp.full((16,), t, jnp.int32)], gv,
                               mask=lane0)
            lloc = g - base
            own = (lloc >= 0) & (lloc < c_pad)
            lclamp = jnp.clip(lloc, 0, c_pad - 1)
            plsc.store_scatter(md_v, [jnp.full((16,), lclamp, jnp.int32)],
                               jnp.full((16,), -1.0, jnp.float32),
                               mask=lane0 & jnp.full((16,), own))
            cxv = jnp.full((16,), cx, jnp.float32)
            cyv = jnp.full((16,), cy, jnp.float32)
            czv = jnp.full((16,), cz, jnp.float32)

            @plsc.parallel_loop(
                0, c_pad, 32, unroll=4,
                carry=(jnp.full((16,), NEG, jnp.float32), lanes))
            def scan(off, car):
                best, bidx = car
                for k in range(2):
                    o = off + k * 16
                    x = pts_v[0, pl.ds(o, 16)]
                    y = pts_v[1, pl.ds(o, 16)]
                    z = pts_v[2, pl.ds(o, 16)]
                    dx = x - cxv
                    dy = y - cyv
                    dz = z - czv
                    d = dx * dx + dy * dy + dz * dz
                    md = md_v[pl.ds(o, 16)]
                    mm = jnp.minimum(md, d)
                    md_v[pl.ds(o, 16)] = mm
                    upd = mm > best
                    best = jnp.where(upd, mm, best)
                    bidx = jnp.where(upd, o + lanes, bidx)
                return best, bidx

            best, bidx = scan

            mx = jnp.max(best)
            eq = best == jnp.full((16,), mx, jnp.float32)
            lane = plsc.all_reduce_ffs(eq)
            lanev = (lane if lane.shape == (16,)
                     else jnp.full((16,), lane, jnp.int32))
            bidxbuf[...] = bidx
            lbest = plsc.load_gather(bidxbuf, [lanev])
            coordsb = plsc.load_gather(pts_v, [rowsel, lbest])
            return publish_combine(mx, coordsb, lbest + base)

        lax.fori_loop(0, m, step, init_carry)

        pltpu.sync_copy(out_v, out_hbm.at[pl.ds(batch * m, m)])

    def wrapper(points):
        pts = jnp.transpose(points, (2, 0, 1))
        pts = jnp.pad(pts, ((0, 0), (0, 0), (0, npad - n)))
        return fps(pts.reshape(-1)).reshape(b, m)

    return wrapper


_fps_cache = None


def kernel(points):
    global _fps_cache
    if _fps_cache is None:
        _fps_cache = _build(B, N, M, C)
    return _fps_cache(points)
